# baseline (speedup 1.0000x reference)
import math

import jax
import jax.numpy as jnp
import numpy as np
from jax import lax
from jax.experimental import pallas as pl
from jax.experimental.pallas import tpu as pltpu
from jax.experimental.pallas import tpu_sc as plsc

N_NODES = 10000
N_EDGES = 160000
IN_DIM = 256
OUT_DIM = 256
H = 8
DH = 32
DE = 16
HALF = 128
TABW = HALF // 2
NC = 2
NS = 16
LANES = 16

EDGES_PER_TILE = N_EDGES // NS
EB = 80
NBLK = EDGES_PER_TILE // EB
NPAD = 10240
ROWS_PER_TILE = NPAD // NS
RCH = 32
NRCH = ROWS_PER_TILE // RCH

BN = 1000
BE_BLK = 2000

_pi = np.empty(HALF, np.int64)
for _h in range(4):
    _pi[16 * _h:16 * _h + 16] = 32 * _h + np.arange(16)
    _pi[64 + 16 * _h:64 + 16 * _h + 16] = 32 * _h + 16 + np.arange(16)
_GPERM = np.concatenate([_pi, HALF + _pi])


def _pack_bf16_pairs(acc):
    lo = lax.bitcast_convert_type(acc[:, :TABW].astype(jnp.bfloat16),
                                  jnp.uint16).astype(jnp.uint32)
    hi = lax.bitcast_convert_type(acc[:, TABW:].astype(jnp.bfloat16),
                                  jnp.uint16).astype(jnp.uint32)
    return lax.bitcast_convert_type(lo | (hi << 16), jnp.int32)


def _qkv_body(x_ref, w_ref, b_ref, q_ref, k_ref, v_ref):
    xb = x_ref[...]
    for m, o_ref in enumerate((q_ref, k_ref, v_ref)):
        acc = lax.dot_general(xb, w_ref[m, 0], (((1,), (1,)), ((), ())),
                              preferred_element_type=jnp.float32)
        o_ref[...] = _pack_bf16_pairs(acc + b_ref[m, 0, 0])


_qkv_call = pl.pallas_call(
    _qkv_body,
    grid=(N_NODES // BN, NC),
    in_specs=[
        pl.BlockSpec((BN, IN_DIM), lambda r, h: (r, 0)),
        pl.BlockSpec((3, 1, HALF, IN_DIM), lambda r, h: (0, h, 0, 0)),
        pl.BlockSpec((3, 1, 1, HALF), lambda r, h: (0, h, 0, 0)),
    ],
    out_specs=[
        pl.BlockSpec((BN, TABW), lambda r, h: (h * (N_NODES // BN) + r, 0))
        for _ in range(3)
    ],
    out_shape=[jax.ShapeDtypeStruct((NC * N_NODES, TABW), jnp.int32)
               for _ in range(3)],
)


def _eproj_body(a_ref, w_ref, b_ref, o_ref):
    acc = lax.dot_general(a_ref[...], w_ref[0], (((1,), (1,)), ((), ())),
                          preferred_element_type=jnp.float32)
    o_ref[...] = (acc + b_ref[0, 0]).astype(jnp.bfloat16)


_eproj_call = pl.pallas_call(
    _eproj_body,
    grid=(N_EDGES // BE_BLK, NC),
    in_specs=[
        pl.BlockSpec((BE_BLK, DE), lambda r, h: (r, 0)),
        pl.BlockSpec((1, HALF, DE), lambda r, h: (h, 0, 0)),
        pl.BlockSpec((1, 1, HALF), lambda r, h: (h, 0, 0)),
    ],
    out_specs=pl.BlockSpec((BE_BLK, HALF),
                           lambda r, h: (h * (N_EDGES // BE_BLK) + r, 0)),
    out_shape=jax.ShapeDtypeStruct((NC * N_EDGES, HALF), jnp.bfloat16),
)


_GATHER_DNUMS = lax.GatherDimensionNumbers(
    offset_dims=(), collapsed_slice_dims=(0,), start_index_map=(0,))


def _lane_permute(t, idx):
    return lax.gather(t, idx[:, None], _GATHER_DNUMS, slice_sizes=(1,),
                      mode=lax.GatherScatterMode.PROMISE_IN_BOUNDS)


_MASK_HI = -65536


def _unpk(w):
    lo = lax.bitcast_convert_type(w << 16, jnp.float32)
    hi = lax.bitcast_convert_type(w & _MASK_HI, jnp.float32)
    return lo, hi


def _sc_body(ktab, qtab, vtab, etab, src_hbm, dst_hbm, out_hbm,
             srcb, dstb, dadj, kb, qb, vb, eb, mb, stage, acc,
             ksem, qsem, vsem, esem):
    cid = lax.axis_index("c")
    sid = lax.axis_index("s")
    zeros16 = jnp.zeros((LANES,), jnp.float32)

    def _zrow(r, carry):
        for j in range(HALF // LANES):
            stage[r, pl.ds(j * LANES, LANES)] = zeros16
        return carry
    lax.fori_loop(0, RCH, _zrow, 0)
    for c in range(NRCH):
        pltpu.sync_copy(stage, acc.at[pl.ds(sid * ROWS_PER_TILE + c * RCH, RCH)])
    plsc.subcore_barrier()

    noff = cid * N_NODES
    eoff = cid * N_EDGES

    def _blk(b, carry):
        base = sid * EDGES_PER_TILE + b * EB
        pltpu.sync_copy(src_hbm.at[pl.ds(base, EB)], srcb)
        pltpu.sync_copy(dst_hbm.at[pl.ds(base, EB)], dstb)
        for j in range(EB // LANES):
            sl = pl.ds(j * LANES, LANES)
            srcb[sl] = srcb[sl] + noff
            dadj[sl] = dstb[sl] + noff
        ck = pltpu.async_copy(ktab.at[srcb], kb, ksem)
        cq = pltpu.async_copy(qtab.at[dadj], qb, qsem)
        cv = pltpu.async_copy(vtab.at[srcb], vb, vsem)
        ce = pltpu.async_copy(etab.at[pl.ds(eoff + base, EB)], eb, esem)
        ck.wait()
        cq.wait()
        cv.wait()
        ce.wait()

        lane = lax.iota(jnp.int32, LANES)

        def _pair_load(ref, row2, col):
            w = ref[row2, pl.ds(col, LANES)].astype(jnp.float32)
            return w[0], w[1]

        def _edge(i2, icarry):
            row2 = pl.ds(i2 * 2, 2)
            ia = i2 * 2
            ib = ia + 1
            sva = []
            svb = []
            for h in range(HALF // DH):
                sw = pl.ds(h * LANES, LANES)
                ka_lo, ka_hi = _unpk(kb[ia, sw])
                kb_lo, kb_hi = _unpk(kb[ib, sw])
                qa_lo, qa_hi = _unpk(qb[ia, sw])
                qb_lo, qb_hi = _unpk(qb[ib, sw])
                ea_lo, eb_lo = _pair_load(eb, row2, LANES * h)
                ea_hi, eb_hi = _pair_load(eb, row2, TABW + LANES * h)
                ta = ka_lo * qa_lo * ea_lo + ka_hi * qa_hi * ea_hi
                tb = kb_lo * qb_lo * eb_lo + kb_hi * qb_hi * eb_hi
                for step in (8, 4, 2, 1):
                    ta = ta + _lane_permute(ta, lane ^ step)
                    tb = tb + _lane_permute(tb, lane ^ step)
                sva.append(jnp.exp(jnp.clip(ta, -5.0, 5.0)))
                svb.append(jnp.exp(jnp.clip(tb, -5.0, 5.0)))
            for i, sv in ((ia, sva), (ib, svb)):
                for j in range(4):
                    v_lo, v_hi = _unpk(vb[i, pl.ds(j * LANES, LANES)])
                    mb[i, pl.ds(j * LANES, LANES)] = v_lo * sv[j // 2]
                    mb[i, pl.ds(TABW + j * LANES, LANES)] = v_hi * sv[2 + j // 2]
            return icarry
        lax.fori_loop(0, EB // 2, _edge, 0)
        pltpu.sync_copy(mb, acc.at[dstb], add=True)
        return carry
    lax.fori_loop(0, NBLK, _blk, 0)

    plsc.subcore_barrier()
    for c in range(NRCH):
        r0 = sid * ROWS_PER_TILE + c * RCH
        pltpu.sync_copy(acc.at[pl.ds(r0, RCH)], stage)
        pltpu.sync_copy(stage, out_hbm.at[pl.ds(cid * NPAD + r0, RCH)])


_sc_mesh = plsc.VectorSubcoreMesh(core_axis_name="c", subcore_axis_name="s",
                                  num_cores=NC, num_subcores=NS)

_sc_call = pl.kernel(
    _sc_body,
    out_type=jax.ShapeDtypeStruct((NC * NPAD, HALF), jnp.float32),
    mesh=_sc_mesh,
    compiler_params=pltpu.CompilerParams(use_tc_tiling_on_sc=False),
    scratch_types=[
        pltpu.VMEM((EB,), jnp.int32),
        pltpu.VMEM((EB,), jnp.int32),
        pltpu.VMEM((EB,), jnp.int32),
        pltpu.VMEM((EB, TABW), jnp.int32),
        pltpu.VMEM((EB, TABW), jnp.int32),
        pltpu.VMEM((EB, TABW), jnp.int32),
        pltpu.VMEM((EB, HALF), jnp.bfloat16),
        pltpu.VMEM((EB, HALF), jnp.float32),
        pltpu.VMEM((RCH, HALF), jnp.float32),
        pltpu.VMEM_SHARED((NPAD, HALF), jnp.float32),
        pltpu.SemaphoreType.DMA,
        pltpu.SemaphoreType.DMA,
        pltpu.SemaphoreType.DMA,
        pltpu.SemaphoreType.DMA,
    ],
)


def kernel(x, expander_edge_index, expander_edge_attr, batch,
           WQ, bQ, WK, bK, WE, bE, WV, bV):
    scale = 1.0 / math.sqrt(DH)
    gp = _GPERM
    w_stack = jnp.stack([WQ[gp] * scale, WK[gp], WV]).reshape(
        3, NC, HALF, IN_DIM)
    b_stack = jnp.stack([bQ[gp] * scale, bK[gp], bV]).reshape(3, NC, 1, HALF)
    qtab, ktab, vtab = _qkv_call(x, w_stack, b_stack)
    etab = _eproj_call(expander_edge_attr, WE[gp].reshape(NC, HALF, DE),
                       bE[gp].reshape(NC, 1, HALF))
    src = expander_edge_index[0]
    dst = expander_edge_index[1]
    out2 = _sc_call(ktab, qtab, vtab, etab, src, dst)
    return (out2.reshape(NC, NPAD, HALF)[:, :N_NODES]
            .transpose(1, 0, 2).reshape(N_NODES, OUT_DIM))


# speedup vs baseline: 1.2246x; 1.2246x over previous
import math

import jax
import jax.numpy as jnp
import numpy as np
from jax import lax
from jax.experimental import pallas as pl
from jax.experimental.pallas import tpu as pltpu
from jax.experimental.pallas import tpu_sc as plsc

N_NODES = 10000
N_EDGES = 160000
IN_DIM = 256
OUT_DIM = 256
H = 8
DH = 32
DE = 16
HALF = 128
TABW = HALF // 2
NC = 2
NS = 16
LANES = 16

EDGES_PER_TILE = N_EDGES // NS
EB = 80
NBLK = EDGES_PER_TILE // EB
NPAD = 10240
ROWS_PER_TILE = NPAD // NS
RCH = 32
NRCH = ROWS_PER_TILE // RCH

BN = 1000
BE_BLK = 2000

_pi = np.empty(HALF, np.int64)
for _h in range(4):
    _pi[16 * _h:16 * _h + 16] = 32 * _h + np.arange(16)
    _pi[64 + 16 * _h:64 + 16 * _h + 16] = 32 * _h + 16 + np.arange(16)
_GPERM = np.concatenate([_pi, HALF + _pi])


def _zip_pack(lo_f32, hi_f32):
    lo = lax.bitcast_convert_type(lo_f32.astype(jnp.bfloat16),
                                  jnp.uint16).astype(jnp.uint32)
    hi = lax.bitcast_convert_type(hi_f32.astype(jnp.bfloat16),
                                  jnp.uint16).astype(jnp.uint32)
    return lax.bitcast_convert_type(lo | (hi << 16), jnp.int32)


def _qkv_body(x_ref, w_ref, b_ref, q_ref, k_ref, v_ref):
    xb = x_ref[...]
    for m, o_ref in enumerate((q_ref, k_ref, v_ref)):
        acc = lax.dot_general(xb, w_ref[m], (((1,), (1,)), ((), ())),
                              preferred_element_type=jnp.float32)
        acc = acc + b_ref[m, 0]
        for half in range(NC):
            base = half * HALF
            o_ref[:, half * TABW:(half + 1) * TABW] = _zip_pack(
                acc[:, base:base + TABW], acc[:, base + TABW:base + HALF])


_qkv_call = pl.pallas_call(
    _qkv_body,
    grid=(N_NODES // BN,),
    in_specs=[
        pl.BlockSpec((BN, IN_DIM), lambda r: (r, 0)),
        pl.BlockSpec((3, OUT_DIM, IN_DIM), lambda r: (0, 0, 0)),
        pl.BlockSpec((3, 1, OUT_DIM), lambda r: (0, 0, 0)),
    ],
    out_specs=[
        pl.BlockSpec((BN, HALF), lambda r: (r, 0))
        for _ in range(3)
    ],
    out_shape=[jax.ShapeDtypeStruct((N_NODES, HALF), jnp.int32)
               for _ in range(3)],
)


def _eproj_body(a_ref, w_ref, b_ref, o_ref):
    acc = lax.dot_general(a_ref[...], w_ref[0], (((1,), (1,)), ((), ())),
                          preferred_element_type=jnp.float32)
    acc = (acc + b_ref[0, 0]).reshape(BE_BLK // 2, 2, HALF)
    o_ref[0] = _zip_pack(acc[:, 0, :], acc[:, 1, :])


_eproj_call = pl.pallas_call(
    _eproj_body,
    grid=(N_EDGES // BE_BLK, NC),
    in_specs=[
        pl.BlockSpec((BE_BLK, DE), lambda r, h: (r, 0)),
        pl.BlockSpec((1, HALF, DE), lambda r, h: (h, 0, 0)),
        pl.BlockSpec((1, 1, HALF), lambda r, h: (h, 0, 0)),
    ],
    out_specs=pl.BlockSpec(
        (1, BE_BLK // 2, HALF),
        lambda r, h: (h, r, 0)),
    out_shape=jax.ShapeDtypeStruct((NC, N_EDGES // 2, HALF), jnp.int32),
)


_GATHER_DNUMS = lax.GatherDimensionNumbers(
    offset_dims=(), collapsed_slice_dims=(0,), start_index_map=(0,))


def _lane_permute(t, idx):
    return lax.gather(t, idx[:, None], _GATHER_DNUMS, slice_sizes=(1,),
                      mode=lax.GatherScatterMode.PROMISE_IN_BOUNDS)


def _unpk(w):
    lo = lax.bitcast_convert_type(w << 16, jnp.float32)
    hi = lax.bitcast_convert_type(w, jnp.float32)
    return lo, hi


def _sc_body(ktab, qtab, vtab, etab, src_hbm, dst_hbm, out_hbm,
             srcb, dstb, dadj, kb, qb, vb, ebp, mb, stage, acc,
             ksem, qsem, vsem, esem):
    cid = lax.axis_index("c")
    sid = lax.axis_index("s")
    zeros16 = jnp.zeros((LANES,), jnp.float32)

    def _zrow(r, carry):
        for j in range(HALF // LANES):
            stage[r, pl.ds(j * LANES, LANES)] = zeros16
        return carry
    lax.fori_loop(0, RCH, _zrow, 0)
    for c in range(NRCH):
        pltpu.sync_copy(stage, acc.at[pl.ds(sid * ROWS_PER_TILE + c * RCH, RCH)])
    plsc.subcore_barrier()

    eoff = cid * (N_EDGES // 2)

    def _blk(b, carry):
        base = sid * EDGES_PER_TILE + b * EB
        pltpu.sync_copy(src_hbm.at[pl.ds(base, EB)], srcb)
        pltpu.sync_copy(dst_hbm.at[pl.ds(base, EB)], dstb)
        for j in range(EB // LANES):
            sl = pl.ds(j * LANES, LANES)
            srcb[sl] = srcb[sl] * 2 + cid
            dadj[sl] = dstb[sl] * 2 + cid
        ck = pltpu.async_copy(ktab.at[srcb], kb, ksem)
        cq = pltpu.async_copy(qtab.at[dadj], qb, qsem)
        cv = pltpu.async_copy(vtab.at[srcb], vb, vsem)
        ce = pltpu.async_copy(etab.at[pl.ds(eoff + base // 2, EB // 2)], ebp,
                              esem)
        ck.wait()
        cq.wait()
        cv.wait()
        ce.wait()

        lane = lax.iota(jnp.int32, LANES)

        def _edge(i2, icarry):
            ia = i2 * 2
            ib = ia + 1
            sva = []
            svb = []
            for h in range(HALF // DH):
                w0 = pl.ds(h * LANES, LANES)
                w1 = pl.ds(TABW + h * LANES, LANES)
                ka_lo, ka_hi = _unpk(kb[ia, w0])
                kb_lo, kb_hi = _unpk(kb[ib, w0])
                qa_lo, qa_hi = _unpk(qb[ia, w0])
                qb_lo, qb_hi = _unpk(qb[ib, w0])
                ea_lo, eb_lo = _unpk(ebp[i2, w0])
                ea_hi, eb_hi = _unpk(ebp[i2, w1])
                ta = ka_lo * qa_lo * ea_lo + ka_hi * qa_hi * ea_hi
                tb = kb_lo * qb_lo * eb_lo + kb_hi * qb_hi * eb_hi
                for step in (8, 4, 2, 1):
                    ta = ta + _lane_permute(ta, lane ^ step)
                    tb = tb + _lane_permute(tb, lane ^ step)
                sva.append(jnp.exp(jnp.clip(ta, -5.0, 5.0)))
                svb.append(jnp.exp(jnp.clip(tb, -5.0, 5.0)))
            for i, sv in ((ia, sva), (ib, svb)):
                for j in range(4):
                    v_lo, v_hi = _unpk(vb[i, pl.ds(j * LANES, LANES)])
                    mb[i, pl.ds(j * LANES, LANES)] = v_lo * sv[j // 2]
                    mb[i, pl.ds(TABW + j * LANES, LANES)] = v_hi * sv[2 + j // 2]
            return icarry
        lax.fori_loop(0, EB // 2, _edge, 0)
        pltpu.sync_copy(mb, acc.at[dstb], add=True)
        return carry
    lax.fori_loop(0, NBLK, _blk, 0)

    plsc.subcore_barrier()
    for c in range(NRCH):
        r0 = sid * ROWS_PER_TILE + c * RCH
        pltpu.sync_copy(acc.at[pl.ds(r0, RCH)], stage)
        pltpu.sync_copy(stage, out_hbm.at[pl.ds(cid * NPAD + r0, RCH)])


_sc_mesh = plsc.VectorSubcoreMesh(core_axis_name="c", subcore_axis_name="s",
                                  num_cores=NC, num_subcores=NS)

_sc_call = pl.kernel(
    _sc_body,
    out_type=jax.ShapeDtypeStruct((NC * NPAD, HALF), jnp.float32),
    mesh=_sc_mesh,
    compiler_params=pltpu.CompilerParams(use_tc_tiling_on_sc=False),
    scratch_types=[
        pltpu.VMEM((EB,), jnp.int32),
        pltpu.VMEM((EB,), jnp.int32),
        pltpu.VMEM((EB,), jnp.int32),
        pltpu.VMEM((EB, TABW), jnp.int32),
        pltpu.VMEM((EB, TABW), jnp.int32),
        pltpu.VMEM((EB, TABW), jnp.int32),
        pltpu.VMEM((EB // 2, HALF), jnp.int32),
        pltpu.VMEM((EB, HALF), jnp.float32),
        pltpu.VMEM((RCH, HALF), jnp.float32),
        pltpu.VMEM_SHARED((NPAD, HALF), jnp.float32),
        pltpu.SemaphoreType.DMA,
        pltpu.SemaphoreType.DMA,
        pltpu.SemaphoreType.DMA,
        pltpu.SemaphoreType.DMA,
    ],
)


def kernel(x, expander_edge_index, expander_edge_attr, batch,
           WQ, bQ, WK, bK, WE, bE, WV, bV):
    scale = 1.0 / math.sqrt(DH)
    gp = _GPERM
    w_stack = jnp.stack([WQ[gp] * scale, WK[gp], WV])
    b_stack = jnp.stack([bQ[gp] * scale, bK[gp], bV]).reshape(3, 1, OUT_DIM)
    qtab, ktab, vtab = _qkv_call(x, w_stack, b_stack)
    etab = _eproj_call(expander_edge_attr, WE[gp].reshape(NC, HALF, DE),
                       bE[gp].reshape(NC, 1, HALF))
    src = expander_edge_index[0]
    dst = expander_edge_index[1]
    out2 = _sc_call(ktab.reshape(2 * N_NODES, TABW),
                    qtab.reshape(2 * N_NODES, TABW),
                    vtab.reshape(2 * N_NODES, TABW),
                    etab.reshape(N_EDGES // 2 * NC, HALF), src, dst)
    return (out2.reshape(NC, NPAD, HALF)[:, :N_NODES]
            .transpose(1, 0, 2).reshape(N_NODES, OUT_DIM))
